# baseline (speedup 1.0000x reference)
import functools

import jax
import jax.numpy as jnp
from jax import lax
from jax.experimental import pallas as pl
from jax.experimental.pallas import tpu as pltpu
from jax.experimental.pallas import tpu_sc as plsc

VOCAB = 1000
NUM_TOKENS = 1024 * 50

NUM_CORES = 2
NUM_SUBCORES = 16
NUM_WORKERS = NUM_CORES * NUM_SUBCORES

VC = 200
NKV = VOCAB // VC
NVT = VC // 8
NTB = 50 * 8
UNITS_PER_TILE = -(-NTB // NUM_WORKERS)


def _matmul_body(e_ref, p_ref, *g_refs):
    d = lax.dot_general(
        e_ref[...], p_ref[...],
        dimension_numbers=(((1,), (1,)), ((), ())),
        preferred_element_type=jnp.float32,
    )
    for k in range(NKV):
        g_refs[k][...] = d[:, k * VC:(k + 1) * VC]


def _compute_g(embed_table, proj_W):
    return pl.pallas_call(
        _matmul_body,
        out_shape=[jax.ShapeDtypeStruct((VOCAB, VC), jnp.float32)] * NKV,
    )(embed_table, proj_W)


def _transpose_chunk(gbuf, tbuf):
    iota = lax.iota(jnp.int32, 16)
    rowidx = [iota + 16 * c for c in range(8)]

    def vbody(vt, carry):
        base = vt * 8
        for vs in range(8):
            col = jnp.full((16,), base + vs, jnp.int32)
            for c in range(8):
                vec = plsc.load_gather(gbuf, [rowidx[c], col])
                tbuf[vt, vs, pl.ds(c * 16, 16)] = vec
        return carry

    lax.fori_loop(0, NVT, vbody, 0)


@functools.partial(
    pl.kernel,
    out_type=jax.ShapeDtypeStruct((50, 125, 8, 8, 128), jnp.float32),
    mesh=plsc.VectorSubcoreMesh(core_axis_name="c", subcore_axis_name="s"),
    scratch_types=[
        pltpu.VMEM((128,), jnp.int32),
        pltpu.VMEM((2, 128, VC), jnp.float32),
        pltpu.VMEM((2, NVT, 8, 128), jnp.float32),
        pltpu.SemaphoreType.DMA,
        pltpu.SemaphoreType.DMA,
        pltpu.SemaphoreType.DMA,
        pltpu.SemaphoreType.DMA,
    ],
    compiler_params=pltpu.CompilerParams(
        use_tc_tiling_on_sc=False, needs_layout_passes=False),
)
def _gather_kernel(idt_hbm, g0, g1, g2, g3, g4, out_hbm,
                   idx_v, gbuf, tbuf, gsem0, gsem1, wsem0, wsem1):
    gk = (g0, g1, g2, g3, g4)
    gsem = (gsem0, gsem1)
    wsem = (wsem0, wsem1)
    wid = lax.axis_index("s") * NUM_CORES + lax.axis_index("c")

    def unit(jj, carry):
        tb = wid + NUM_WORKERS * jj

        @pl.when(tb < NTB)
        def _():
            t = tb // 8
            bb = tb % 8
            pltpu.sync_copy(idt_hbm.at[pl.ds(tb * 128, 128)], idx_v)
            gd = [None] * NKV
            wd = [None] * NKV
            gd[0] = pltpu.async_copy(gk[0].at[idx_v], gbuf.at[0], gsem[0])
            for kv in range(NKV):
                p = kv & 1
                if kv + 1 < NKV:
                    gd[kv + 1] = pltpu.async_copy(
                        gk[kv + 1].at[idx_v], gbuf.at[1 - p], gsem[1 - p])
                gd[kv].wait()
                if kv >= 2:
                    wd[kv - 2].wait()
                _transpose_chunk(gbuf.at[p], tbuf.at[p])
                wd[kv] = pltpu.async_copy(
                    tbuf.at[p], out_hbm.at[t, pl.ds(kv * NVT, NVT), bb],
                    wsem[p])
            wd[NKV - 2].wait()
            wd[NKV - 1].wait()

        return carry

    lax.fori_loop(0, UNITS_PER_TILE, unit, 0)


def kernel(input_ids, embed_table, proj_W):
    b, t = input_ids.shape
    ids_t = input_ids.T.reshape(-1).astype(jnp.int32)
    gs = _compute_g(embed_table, proj_W)
    out5 = _gather_kernel(ids_t, *gs)
    return out5.transpose(2, 4, 0, 1, 3).reshape(b, t, VOCAB)


# speedup vs baseline: 3.3069x; 3.3069x over previous
import functools

import jax
import jax.numpy as jnp
from jax import lax
from jax.experimental import pallas as pl
from jax.experimental.pallas import tpu as pltpu
from jax.experimental.pallas import tpu_sc as plsc

VOCAB = 1000
D_MODEL = 64
B_TOK = 1024
T_SEQ = 50

NUM_CORES = 2
NUM_SUBCORES = 16
NUM_WORKERS = NUM_CORES * NUM_SUBCORES

NTB = T_SEQ * 8
UNITS_PER_TILE = -(-NTB // NUM_WORKERS)


def _transpose_unit(wbuf, tbuf):
    iota = lax.iota(jnp.int32, 16)
    rowidx = [iota + 16 * c for c in range(8)]

    def dbody(dt, carry):
        base = dt * 8
        for ds in range(8):
            col = jnp.full((16,), base + ds, jnp.int32)
            for c in range(8):
                vec = plsc.load_gather(wbuf, [rowidx[c], col])
                tbuf[dt, ds, pl.ds(c * 16, 16)] = vec
        return carry

    lax.fori_loop(0, 8, dbody, 0)


@functools.partial(
    pl.kernel,
    out_type=jax.ShapeDtypeStruct((T_SEQ, 8, 8, 8, 128), jnp.float32),
    mesh=plsc.VectorSubcoreMesh(core_axis_name="c", subcore_axis_name="s"),
    scratch_types=[
        pltpu.VMEM((128,), jnp.int32),
        pltpu.VMEM((2, 128, D_MODEL), jnp.float32),
        pltpu.VMEM((2, 8, 8, 128), jnp.float32),
        pltpu.SemaphoreType.DMA,
        pltpu.SemaphoreType.DMA,
        pltpu.SemaphoreType.DMA,
        pltpu.SemaphoreType.DMA,
    ],
    compiler_params=pltpu.CompilerParams(
        use_tc_tiling_on_sc=False, needs_layout_passes=False),
)
def _gather_kernel(idt_hbm, emb_hbm, out_hbm,
                   idx_v, wbuf, tbuf, gsem0, gsem1, wsem0, wsem1):
    gsem = (gsem0, gsem1)
    wsem = (wsem0, wsem1)
    wid = lax.axis_index("s") * NUM_CORES + lax.axis_index("c")

    def unit(jj, carry):
        tb = wid + NUM_WORKERS * jj

        @pl.when(tb < NTB)
        def _():
            t = tb // 8
            bb = tb % 8
            pltpu.sync_copy(idt_hbm.at[pl.ds(tb * 128, 128)], idx_v)
            g0 = pltpu.async_copy(emb_hbm.at[idx_v], wbuf.at[0], gsem[0])
            g0.wait()
            _transpose_unit(wbuf.at[0], tbuf.at[0])
            pltpu.sync_copy(tbuf.at[0], out_hbm.at[t, :, bb])

        return carry

    lax.fori_loop(0, UNITS_PER_TILE, unit, 0)


def _matmul_body(x_ref, p_ref, o_ref):
    o_ref[0] = lax.dot_general(
        p_ref[...], x_ref[0],
        dimension_numbers=(((1,), (0,)), ((), ())),
        preferred_element_type=jnp.float32,
    )


def _project(xt, proj_W):
    return pl.pallas_call(
        _matmul_body,
        grid=(T_SEQ,),
        in_specs=[
            pl.BlockSpec((1, D_MODEL, B_TOK), lambda t: (t, 0, 0)),
            pl.BlockSpec((VOCAB, D_MODEL), lambda t: (0, 0)),
        ],
        out_specs=pl.BlockSpec((1, VOCAB, B_TOK), lambda t: (t, 0, 0)),
        out_shape=jax.ShapeDtypeStruct((T_SEQ, VOCAB, B_TOK), jnp.float32),
    )(xt, proj_W)


def kernel(input_ids, embed_table, proj_W):
    b, t = input_ids.shape
    ids_t = input_ids.T.reshape(-1).astype(jnp.int32)
    x5 = _gather_kernel(ids_t, embed_table)
    xt = x5.transpose(0, 1, 3, 2, 4).reshape(T_SEQ, D_MODEL, B_TOK)
    out3 = _project(xt, proj_W)
    return out3.transpose(2, 0, 1).reshape(b, t, VOCAB)


# speedup vs baseline: 3.4632x; 1.0472x over previous
import functools

import jax
import jax.numpy as jnp
from jax import lax
from jax.experimental import pallas as pl
from jax.experimental.pallas import tpu as pltpu
from jax.experimental.pallas import tpu_sc as plsc

VOCAB = 1000
D_MODEL = 64
B_TOK = 1024
T_SEQ = 50

NUM_CORES = 2
NUM_SUBCORES = 16
NUM_WORKERS = NUM_CORES * NUM_SUBCORES

NTB = T_SEQ * 8
FULL_UNITS = NTB // NUM_WORKERS
TAIL_UNITS = NTB - FULL_UNITS * NUM_WORKERS


def _transpose_unit(wbuf, tbuf):
    iota = lax.iota(jnp.int32, 16)
    rowidx = [iota + 16 * c for c in range(8)]

    def dbody(dt, carry):
        base = dt * 8
        for ds in range(8):
            col = jnp.full((16,), base + ds, jnp.int32)
            for c in range(8):
                vec = plsc.load_gather(wbuf, [rowidx[c], col])
                tbuf[dt, ds, pl.ds(c * 16, 16)] = vec
        return carry

    lax.fori_loop(0, 8, dbody, 0)


@functools.partial(
    pl.kernel,
    out_type=jax.ShapeDtypeStruct((T_SEQ, 8, 8, 8, 128), jnp.float32),
    mesh=plsc.VectorSubcoreMesh(core_axis_name="c", subcore_axis_name="s"),
    scratch_types=[
        pltpu.VMEM(((FULL_UNITS + 1) * 128,), jnp.int32),
        pltpu.VMEM((2, 128, D_MODEL), jnp.float32),
        pltpu.VMEM((2, 8, 8, 128), jnp.float32),
        pltpu.SemaphoreType.DMA,
        pltpu.SemaphoreType.DMA,
        pltpu.SemaphoreType.DMA,
        pltpu.SemaphoreType.DMA,
    ],
    compiler_params=pltpu.CompilerParams(
        use_tc_tiling_on_sc=False, needs_layout_passes=False),
)
def _gather_kernel(idt_hbm, emb_hbm, out_hbm,
                   idx_v, wbuf, tbuf, gsem0, gsem1, wsem0, wsem1):
    gsem = (gsem0, gsem1)
    wsem = (wsem0, wsem1)
    wid = lax.axis_index("s") * NUM_CORES + lax.axis_index("c")
    base_tb = wid * FULL_UNITS
    pltpu.sync_copy(idt_hbm.at[pl.ds(base_tb * 128, FULL_UNITS * 128)],
                    idx_v.at[pl.ds(0, FULL_UNITS * 128)])

    def idx_slice(j):
        return idx_v.at[pl.ds(j * 128, 128)]

    def dst(tb):
        return out_hbm.at[tb // 8, :, tb % 8]

    gd = [None] * FULL_UNITS
    wd = [None] * FULL_UNITS
    gd[0] = pltpu.async_copy(emb_hbm.at[idx_slice(0)], wbuf.at[0], gsem[0])
    for j in range(FULL_UNITS):
        p = j & 1
        if j + 1 < FULL_UNITS:
            gd[j + 1] = pltpu.async_copy(
                emb_hbm.at[idx_slice(j + 1)], wbuf.at[1 - p], gsem[1 - p])
        gd[j].wait()
        if j >= 2:
            wd[j - 2].wait()
        _transpose_unit(wbuf.at[p], tbuf.at[p])
        wd[j] = pltpu.async_copy(tbuf.at[p], dst(base_tb + j), wsem[p])
    wd[FULL_UNITS - 2].wait()
    wd[FULL_UNITS - 1].wait()

    @pl.when(wid < TAIL_UNITS)
    def _():
        tb = NUM_WORKERS * FULL_UNITS + wid
        pltpu.sync_copy(idt_hbm.at[pl.ds(tb * 128, 128)],
                        idx_v.at[pl.ds(FULL_UNITS * 128, 128)])
        g = pltpu.async_copy(
            emb_hbm.at[idx_v.at[pl.ds(FULL_UNITS * 128, 128)]],
            wbuf.at[0], gsem[0])
        g.wait()
        _transpose_unit(wbuf.at[0], tbuf.at[0])
        pltpu.sync_copy(tbuf.at[0], dst(tb))


def _matmul_body(x_ref, p_ref, o_ref):
    o_ref[0] = lax.dot_general(
        p_ref[...], x_ref[0],
        dimension_numbers=(((1,), (0,)), ((), ())),
        preferred_element_type=jnp.float32,
    )


def _project(xt, proj_W):
    return pl.pallas_call(
        _matmul_body,
        grid=(T_SEQ,),
        in_specs=[
            pl.BlockSpec((1, D_MODEL, B_TOK), lambda t: (t, 0, 0)),
            pl.BlockSpec((VOCAB, D_MODEL), lambda t: (0, 0)),
        ],
        out_specs=pl.BlockSpec((1, VOCAB, B_TOK), lambda t: (t, 0, 0)),
        out_shape=jax.ShapeDtypeStruct((T_SEQ, VOCAB, B_TOK), jnp.float32),
    )(xt, proj_W)


def kernel(input_ids, embed_table, proj_W):
    b, t = input_ids.shape
    ids_t = input_ids.T.reshape(-1).astype(jnp.int32)
    x5 = _gather_kernel(ids_t, embed_table)
    xt = x5.transpose(0, 1, 3, 2, 4).reshape(T_SEQ, D_MODEL, B_TOK)
    out3 = _project(xt, proj_W)
    return out3.transpose(2, 0, 1).reshape(b, t, VOCAB)


# speedup vs baseline: 3.9757x; 1.1480x over previous
import functools

import jax
import jax.numpy as jnp
from jax import lax
from jax.experimental import pallas as pl
from jax.experimental.pallas import tpu as pltpu
from jax.experimental.pallas import tpu_sc as plsc

VOCAB = 1000
D_MODEL = 64
B_TOK = 1024
T_SEQ = 50

NUM_CORES = 2
NUM_SUBCORES = 16
NUM_WORKERS = NUM_CORES * NUM_SUBCORES

NTB = T_SEQ * 8
FULL_UNITS = NTB // NUM_WORKERS
TAIL_UNITS = NTB - FULL_UNITS * NUM_WORKERS


def _transpose_unit(wbuf, tbuf):
    iota = lax.iota(jnp.int32, 16)
    rowidx = [iota + 16 * c for c in range(8)]

    def dbody(dt, carry):
        base = dt * 8
        for ds in range(8):
            col = jnp.full((16,), base + ds, jnp.int32)
            vecs = [plsc.load_gather(wbuf, [rowidx[c], col])
                    for c in range(8)]
            for c in range(8):
                tbuf[dt, ds, pl.ds(c * 16, 16)] = vecs[c]
        return carry

    lax.fori_loop(0, 8, dbody, 0)


@functools.partial(
    pl.kernel,
    out_type=jax.ShapeDtypeStruct((T_SEQ, 8, 8, 8, 128), jnp.float32),
    mesh=plsc.VectorSubcoreMesh(core_axis_name="c", subcore_axis_name="s"),
    scratch_types=[
        pltpu.VMEM(((FULL_UNITS + 1) * 128,), jnp.int32),
        pltpu.VMEM((2, 128, D_MODEL), jnp.float32),
        pltpu.VMEM((2, 8, 8, 128), jnp.float32),
        pltpu.SemaphoreType.DMA,
        pltpu.SemaphoreType.DMA,
        pltpu.SemaphoreType.DMA,
        pltpu.SemaphoreType.DMA,
    ],
    compiler_params=pltpu.CompilerParams(
        use_tc_tiling_on_sc=False, needs_layout_passes=False),
)
def _gather_kernel(idt_hbm, emb_hbm, out_hbm,
                   idx_v, wbuf, tbuf, gsem0, gsem1, wsem0, wsem1):
    gsem = (gsem0, gsem1)
    wsem = (wsem0, wsem1)
    wid = lax.axis_index("s") * NUM_CORES + lax.axis_index("c")
    base_tb = wid * FULL_UNITS
    pltpu.sync_copy(idt_hbm.at[pl.ds(base_tb * 128, FULL_UNITS * 128)],
                    idx_v.at[pl.ds(0, FULL_UNITS * 128)])

    def idx_slice(j):
        return idx_v.at[pl.ds(j * 128, 128)]

    def dst(tb):
        return out_hbm.at[tb // 8, :, tb % 8]

    gd = [None] * FULL_UNITS
    wd = [None] * FULL_UNITS
    gd[0] = pltpu.async_copy(emb_hbm.at[idx_slice(0)], wbuf.at[0], gsem[0])
    for j in range(FULL_UNITS):
        p = j & 1
        if j + 1 < FULL_UNITS:
            gd[j + 1] = pltpu.async_copy(
                emb_hbm.at[idx_slice(j + 1)], wbuf.at[1 - p], gsem[1 - p])
        gd[j].wait()
        if j >= 2:
            wd[j - 2].wait()
        _transpose_unit(wbuf.at[p], tbuf.at[p])
        wd[j] = pltpu.async_copy(tbuf.at[p], dst(base_tb + j), wsem[p])
    wd[FULL_UNITS - 2].wait()
    wd[FULL_UNITS - 1].wait()

    @pl.when(wid < TAIL_UNITS)
    def _():
        tb = NUM_WORKERS * FULL_UNITS + wid
        pltpu.sync_copy(idt_hbm.at[pl.ds(tb * 128, 128)],
                        idx_v.at[pl.ds(FULL_UNITS * 128, 128)])
        g = pltpu.async_copy(
            emb_hbm.at[idx_v.at[pl.ds(FULL_UNITS * 128, 128)]],
            wbuf.at[0], gsem[0])
        g.wait()
        _transpose_unit(wbuf.at[0], tbuf.at[0])
        pltpu.sync_copy(tbuf.at[0], dst(tb))


def _matmul_body(x_ref, p_ref, o_ref):
    o_ref[0] = lax.dot_general(
        p_ref[...], x_ref[0],
        dimension_numbers=(((1,), (0,)), ((), ())),
        preferred_element_type=jnp.float32,
    )


def _project(xt, proj_W):
    return pl.pallas_call(
        _matmul_body,
        grid=(T_SEQ,),
        in_specs=[
            pl.BlockSpec((1, D_MODEL, B_TOK), lambda t: (t, 0, 0)),
            pl.BlockSpec((VOCAB, D_MODEL), lambda t: (0, 0)),
        ],
        out_specs=pl.BlockSpec((1, VOCAB, B_TOK), lambda t: (t, 0, 0)),
        out_shape=jax.ShapeDtypeStruct((T_SEQ, VOCAB, B_TOK), jnp.float32),
    )(xt, proj_W)


def kernel(input_ids, embed_table, proj_W):
    b, t = input_ids.shape
    ids_t = input_ids.T.reshape(-1).astype(jnp.int32)
    x5 = _gather_kernel(ids_t, embed_table)
    xt = x5.transpose(0, 1, 3, 2, 4).reshape(T_SEQ, D_MODEL, B_TOK)
    out3 = _project(xt, proj_W)
    return out3.transpose(2, 0, 1).reshape(b, t, VOCAB)


# speedup vs baseline: 4.4261x; 1.1133x over previous
import functools

import jax
import jax.numpy as jnp
from jax import lax
from jax.experimental import pallas as pl
from jax.experimental.pallas import tpu as pltpu
from jax.experimental.pallas import tpu_sc as plsc

VOCAB = 1000
D_MODEL = 64
B_TOK = 1024
T_SEQ = 50
T_CHUNK = 25

NUM_CORES = 2
NUM_SUBCORES = 16
NUM_WORKERS = NUM_CORES * NUM_SUBCORES

NTB_C = T_CHUNK * 8
FULL_UNITS = NTB_C // NUM_WORKERS
TAIL_UNITS = NTB_C - FULL_UNITS * NUM_WORKERS


def _transpose_unit(wbuf, tbuf):
    iota = lax.iota(jnp.int32, 16)
    rowidx = [iota + 16 * c for c in range(8)]

    def dbody(dt, carry):
        base = dt * 8
        for ds in range(8):
            col = jnp.full((16,), base + ds, jnp.int32)
            vecs = [plsc.load_gather(wbuf, [rowidx[c], col])
                    for c in range(8)]
            for c in range(8):
                tbuf[dt, ds, pl.ds(c * 16, 16)] = vecs[c]
        return carry

    lax.fori_loop(0, 8, dbody, 0)


def _make_gather(chunk):
    unit_offset = chunk * NTB_C

    @functools.partial(
        pl.kernel,
        out_type=jax.ShapeDtypeStruct((T_CHUNK, 8, 8, 8, 128), jnp.float32),
        mesh=plsc.VectorSubcoreMesh(core_axis_name="c", subcore_axis_name="s"),
        scratch_types=[
            pltpu.VMEM(((FULL_UNITS + 1) * 128,), jnp.int32),
            pltpu.VMEM((2, 128, D_MODEL), jnp.float32),
            pltpu.VMEM((2, 8, 8, 128), jnp.float32),
            pltpu.SemaphoreType.DMA,
            pltpu.SemaphoreType.DMA,
            pltpu.SemaphoreType.DMA,
            pltpu.SemaphoreType.DMA,
        ],
        compiler_params=pltpu.CompilerParams(
            use_tc_tiling_on_sc=False, needs_layout_passes=False),
        name=f"embed_gather_c{chunk}",
    )
    def _gather_kernel(idt_hbm, emb_hbm, out_hbm,
                       idx_v, wbuf, tbuf, gsem0, gsem1, wsem0, wsem1):
        gsem = (gsem0, gsem1)
        wsem = (wsem0, wsem1)
        wid = lax.axis_index("s") * NUM_CORES + lax.axis_index("c")
        base_tb = wid * FULL_UNITS
        pltpu.sync_copy(
            idt_hbm.at[pl.ds((unit_offset + base_tb) * 128,
                             FULL_UNITS * 128)],
            idx_v.at[pl.ds(0, FULL_UNITS * 128)])

        def idx_slice(j):
            return idx_v.at[pl.ds(j * 128, 128)]

        def dst(tb):
            return out_hbm.at[tb // 8, :, tb % 8]

        gd = [None] * FULL_UNITS
        wd = [None] * FULL_UNITS
        gd[0] = pltpu.async_copy(emb_hbm.at[idx_slice(0)], wbuf.at[0],
                                 gsem[0])
        for j in range(FULL_UNITS):
            p = j & 1
            if j + 1 < FULL_UNITS:
                gd[j + 1] = pltpu.async_copy(
                    emb_hbm.at[idx_slice(j + 1)], wbuf.at[1 - p],
                    gsem[1 - p])
            gd[j].wait()
            if j >= 2:
                wd[j - 2].wait()
            _transpose_unit(wbuf.at[p], tbuf.at[p])
            wd[j] = pltpu.async_copy(tbuf.at[p], dst(base_tb + j), wsem[p])
        wd[FULL_UNITS - 2].wait()
        wd[FULL_UNITS - 1].wait()

        @pl.when(wid < TAIL_UNITS)
        def _():
            tb = NUM_WORKERS * FULL_UNITS + wid
            pltpu.sync_copy(
                idt_hbm.at[pl.ds((unit_offset + tb) * 128, 128)],
                idx_v.at[pl.ds(FULL_UNITS * 128, 128)])
            g = pltpu.async_copy(
                emb_hbm.at[idx_v.at[pl.ds(FULL_UNITS * 128, 128)]],
                wbuf.at[0], gsem[0])
            g.wait()
            _transpose_unit(wbuf.at[0], tbuf.at[0])
            pltpu.sync_copy(tbuf.at[0], dst(tb))

    return _gather_kernel


_GATHERS = [_make_gather(0), _make_gather(1)]


def _matmul_body(x_ref, p_ref, o_ref):
    o_ref[0] = lax.dot_general(
        p_ref[...], x_ref[0],
        dimension_numbers=(((1,), (0,)), ((), ())),
        preferred_element_type=jnp.float32,
    )


def _matmul_body2(x_ref, p_ref, big_ref, o_ref):
    o_ref[0] = lax.dot_general(
        p_ref[...], x_ref[0],
        dimension_numbers=(((1,), (0,)), ((), ())),
        preferred_element_type=jnp.float32,
    )


def _project_chunk1(xt, proj_W):
    return pl.pallas_call(
        _matmul_body,
        grid=(T_CHUNK,),
        in_specs=[
            pl.BlockSpec((1, D_MODEL, B_TOK), lambda t: (t, 0, 0)),
            pl.BlockSpec((VOCAB, D_MODEL), lambda t: (0, 0)),
        ],
        out_specs=pl.BlockSpec((1, VOCAB, B_TOK), lambda t: (t, 0, 0)),
        out_shape=jax.ShapeDtypeStruct((T_SEQ, VOCAB, B_TOK), jnp.float32),
    )(xt, proj_W)


def _project_chunk2(xt, proj_W, big):
    return pl.pallas_call(
        _matmul_body2,
        grid=(T_CHUNK,),
        in_specs=[
            pl.BlockSpec((1, D_MODEL, B_TOK), lambda t: (t, 0, 0)),
            pl.BlockSpec((VOCAB, D_MODEL), lambda t: (0, 0)),
            pl.BlockSpec((1, 8, 128), lambda t: (0, 0, 0)),
        ],
        out_specs=pl.BlockSpec((1, VOCAB, B_TOK),
                               lambda t: (t + T_CHUNK, 0, 0)),
        out_shape=jax.ShapeDtypeStruct((T_SEQ, VOCAB, B_TOK), jnp.float32),
        input_output_aliases={2: 0},
    )(xt, proj_W, big)


def kernel(input_ids, embed_table, proj_W):
    b, t = input_ids.shape
    ids_t = input_ids.T.reshape(-1).astype(jnp.int32)
    x5_0 = _GATHERS[0](ids_t, embed_table)
    x5_1 = _GATHERS[1](ids_t, embed_table)
    xt0 = x5_0.transpose(0, 1, 3, 2, 4).reshape(T_CHUNK, D_MODEL, B_TOK)
    xt1 = x5_1.transpose(0, 1, 3, 2, 4).reshape(T_CHUNK, D_MODEL, B_TOK)
    big = _project_chunk1(xt0, proj_W)
    out3 = _project_chunk2(xt1, proj_W, big)
    return out3.transpose(2, 0, 1).reshape(b, t, VOCAB)
